# baseline (speedup 1.0000x reference)
import jax
import jax.numpy as jnp
from jax import lax
from jax.experimental import pallas as pl
from jax.experimental.pallas import tpu as pltpu
from jax.experimental.pallas import tpu_sc as plsc

N_NODES = 10000
N_EDGES = 320000
H = 128
R = 8

NC = 2
NS = 16
NW = NC * NS
CHUNK = 128
CHUNKS_PER_TILE = 79
EDGES_PER_TILE = CHUNKS_PER_TILE * CHUNK
E_PAD = EDGES_PER_TILE * NW
ACC_ROWS = 10240
ROWS_PER_TILE = ACC_ROWS // NS
NBUF = 6


def _xw_body(wc_ref, bases_ref, h_ref, out_ref):
    r = pl.program_id(0)
    w = (wc_ref[r, 0] * bases_ref[0] + wc_ref[r, 1] * bases_ref[1]
         + wc_ref[r, 2] * bases_ref[2] + wc_ref[r, 3] * bases_ref[3])
    out_ref[0] = jnp.dot(h_ref[...], w,
                         preferred_element_type=jnp.float32).astype(jnp.bfloat16)


def _flat_body(src_ref, type_ref, out_ref):
    out_ref[...] = type_ref[...] * N_NODES + src_ref[...]


def _norm_body(pacc_ref, pdeg_ref, out_ref):
    d = jnp.sum(pdeg_ref[:, :N_NODES], axis=0)[:, None]
    inv = 1.0 / jnp.maximum(d, 1.0)
    p = (pacc_ref[0, :N_NODES, :].astype(jnp.float32)
         + pacc_ref[1, :N_NODES, :].astype(jnp.float32))
    out_ref[...] = p * inv


def _sc_body(xw_hbm, flat_hbm, dst_hbm, zacc_hbm, zdeg_hbm,
             pacc_hbm, pdeg_hbm,
             fidx_v, dstx_v, rows_a, rows_b, rows_c, rows_d, rows_e, rows_f,
             deg_local, acc_sh, sem_a, sem_b, sem_c, sem_d, sem_e, sem_f):
    c = lax.axis_index("c")
    s = lax.axis_index("s")
    row0 = s * ROWS_PER_TILE
    bufs = (rows_a, rows_b, rows_c, rows_d, rows_e, rows_f)
    sems = (sem_a, sem_b, sem_c, sem_d, sem_e, sem_f)
    ones16 = jnp.full((16,), 1.0, jnp.float32)
    cp_f = pltpu.async_copy(flat_hbm.at[c, s], fidx_v, sem_a)
    cp_d = pltpu.async_copy(dst_hbm.at[c, s], dstx_v, sem_b)
    pltpu.sync_copy(zacc_hbm, rows_a)
    pltpu.sync_copy(zdeg_hbm, deg_local)
    for j in range(ROWS_PER_TILE // CHUNK):
        pltpu.sync_copy(rows_a, acc_sh.at[pl.ds(row0 + j * CHUNK, CHUNK)])
    cp_f.wait()
    cp_d.wait()
    plsc.subcore_barrier()

    for k in range(NBUF):
        pltpu.async_copy(xw_hbm.at[fidx_v.at[k]], bufs[k], sems[k])

    @pl.loop(0, CHUNKS_PER_TILE, step=NBUF)
    def _round(i):
        for k in range(NBUF):
            def _part(k=k):
                j = i + k
                buf, sem = bufs[k], sems[k]
                pltpu.make_async_copy(xw_hbm.at[fidx_v.at[j]], buf, sem).wait()
                pltpu.sync_copy(buf, acc_sh.at[dstx_v.at[j]], add=True)

                for l in range(CHUNK // 16):
                    idx16 = dstx_v[j, pl.ds(l * 16, 16)]
                    plsc.addupdate_scatter(deg_local, [idx16], ones16)

                @pl.when(j + NBUF < CHUNKS_PER_TILE)
                def _next():
                    pltpu.async_copy(xw_hbm.at[fidx_v.at[j + NBUF]], buf, sem)

            if k == 0:
                _part()
            else:
                pl.when(i + k < CHUNKS_PER_TILE)(_part)

    plsc.subcore_barrier()

    out_off = c * ACC_ROWS + row0
    for j in range(ROWS_PER_TILE // CHUNK):
        pltpu.sync_copy(acc_sh.at[pl.ds(row0 + j * CHUNK, CHUNK)], bufs[j % NBUF])
        pltpu.sync_copy(bufs[j % NBUF], pacc_hbm.at[pl.ds(out_off + j * CHUNK, CHUNK)])
    pltpu.sync_copy(deg_local, pdeg_hbm.at[c * NS + s])


def kernel(edge_index, edge_type, embed_table, bases1, w_comp1, bases2, w_comp2):
    f32 = jnp.float32
    bf16 = jnp.bfloat16
    i32 = jnp.int32
    src = edge_index[0].astype(i32)
    dst = edge_index[1].astype(i32)
    et = edge_type.astype(i32)
    pad = E_PAD - N_EDGES
    src_p = jnp.concatenate([src, jnp.zeros((pad,), i32)]).reshape(E_PAD // 128, 128)
    et_p = jnp.concatenate([et, jnp.zeros((pad,), i32)]).reshape(E_PAD // 128, 128)
    dst_p = jnp.concatenate([dst, jnp.full((pad,), N_NODES, i32)]).reshape(
        NC, NS, CHUNKS_PER_TILE, CHUNK)

    flat = pl.pallas_call(
        _flat_body,
        grid=(),
        in_specs=[pl.BlockSpec((E_PAD // 128, 128), lambda: (0, 0)),
                  pl.BlockSpec((E_PAD // 128, 128), lambda: (0, 0))],
        out_specs=pl.BlockSpec((E_PAD // 128, 128), lambda: (0, 0)),
        out_shape=jax.ShapeDtypeStruct((E_PAD // 128, 128), i32),
    )(src_p, et_p).reshape(NC, NS, CHUNKS_PER_TILE, CHUNK)

    xw = pl.pallas_call(
        _xw_body,
        grid=(R,),
        in_specs=[pl.BlockSpec(memory_space=pltpu.SMEM),
                  pl.BlockSpec((4, H, H), lambda r: (0, 0, 0)),
                  pl.BlockSpec((N_NODES, H), lambda r: (0, 0))],
        out_specs=pl.BlockSpec((1, N_NODES, H), lambda r: (r, 0, 0)),
        out_shape=jax.ShapeDtypeStruct((R, N_NODES, H), bf16),
    )(w_comp2, bases2, embed_table).reshape(R * N_NODES, H)

    zacc = jnp.zeros((CHUNK, H), bf16)
    zdeg = jnp.zeros((ACC_ROWS,), f32)

    mesh = plsc.VectorSubcoreMesh(core_axis_name="c", subcore_axis_name="s")
    pacc, pdeg = pl.kernel(
        _sc_body,
        out_type=(jax.ShapeDtypeStruct((NC * ACC_ROWS, H), bf16),
                  jax.ShapeDtypeStruct((NC * NS, ACC_ROWS), f32)),
        mesh=mesh,
        compiler_params=pltpu.CompilerParams(use_tc_tiling_on_sc=False,
                                             needs_layout_passes=False),
        scratch_types=[
            pltpu.VMEM((CHUNKS_PER_TILE, CHUNK), i32),
            pltpu.VMEM((CHUNKS_PER_TILE, CHUNK), i32),
            pltpu.VMEM((CHUNK, H), bf16),
            pltpu.VMEM((CHUNK, H), bf16),
            pltpu.VMEM((CHUNK, H), bf16),
            pltpu.VMEM((CHUNK, H), bf16),
            pltpu.VMEM((CHUNK, H), bf16),
            pltpu.VMEM((CHUNK, H), bf16),
            pltpu.VMEM((ACC_ROWS,), f32),
            pltpu.VMEM_SHARED((ACC_ROWS, H), bf16),
            pltpu.SemaphoreType.DMA,
            pltpu.SemaphoreType.DMA,
            pltpu.SemaphoreType.DMA,
            pltpu.SemaphoreType.DMA,
            pltpu.SemaphoreType.DMA,
            pltpu.SemaphoreType.DMA,
        ],
    )(xw, flat, dst_p, zacc, zdeg)

    pacc = pacc.reshape(NC, ACC_ROWS, H)

    out = pl.pallas_call(
        _norm_body,
        grid=(),
        in_specs=[pl.BlockSpec((NC, ACC_ROWS, H), lambda: (0, 0, 0)),
                  pl.BlockSpec((NC * NS, ACC_ROWS), lambda: (0, 0))],
        out_specs=pl.BlockSpec((N_NODES, H), lambda: (0, 0)),
        out_shape=jax.ShapeDtypeStruct((N_NODES, H), f32),
    )(pacc, pdeg)
    return out


# speedup vs baseline: 1.2680x; 1.2680x over previous
import jax
import jax.numpy as jnp
from jax import lax
from jax.experimental import pallas as pl
from jax.experimental.pallas import tpu as pltpu
from jax.experimental.pallas import tpu_sc as plsc

N_NODES = 10000
N_EDGES = 320000
H = 128
HC = H // 2
R = 8

NC = 2
NS = 16
CHUNK = 128
CHUNKS_PER_TILE = 157
EDGES_PER_TILE = CHUNKS_PER_TILE * CHUNK
E_PAD = EDGES_PER_TILE * NS
ACC_ROWS = 10240
ROWS_PER_TILE = ACC_ROWS // NS
DEG_W = 16


def _xw_body(wc_ref, bases_ref, h_ref, out_ref):
    r = pl.program_id(0)
    w = (wc_ref[r, 0] * bases_ref[0] + wc_ref[r, 1] * bases_ref[1]
         + wc_ref[r, 2] * bases_ref[2] + wc_ref[r, 3] * bases_ref[3])
    out_ref[0] = jnp.dot(h_ref[...], w, preferred_element_type=jnp.float32)


def _flat_body(src_ref, type_ref, out_ref):
    flat2 = (type_ref[...] * N_NODES + src_ref[...]) * 2
    out_ref[: E_PAD // 128] = flat2
    out_ref[E_PAD // 128 :] = flat2 + 1


def _norm_body(pacc_ref, pdeg_ref, out_ref):
    d = jnp.sum(pdeg_ref[:, :N_NODES], axis=0)[:, None]
    inv = 1.0 / jnp.maximum(d, 1.0)
    out_ref[:, :HC] = pacc_ref[0, :N_NODES, :] * inv
    out_ref[:, HC:] = pacc_ref[1, :N_NODES, :] * inv


def _sc_body(xw_hbm, flat_hbm, dst_hbm, zacc_hbm, zdeg_hbm,
             pacc_hbm, pdeg_hbm,
             fidx_v, dstx_v, rows_a, rows_b, rows_c, rows_d,
             deg_local, acc_sh, sem_a, sem_b, sem_c, sem_d, sem_s):
    c = lax.axis_index("c")
    s = lax.axis_index("s")
    row0 = s * ROWS_PER_TILE
    bufs = (rows_a, rows_b, rows_c, rows_d)
    sems = (sem_a, sem_b, sem_c, sem_d)
    nbuf = len(bufs)
    ones16 = jnp.full((16,), 1.0, jnp.float32)
    cp_f = pltpu.async_copy(flat_hbm.at[c, s], fidx_v, sem_a)
    cp_d = pltpu.async_copy(dst_hbm.at[s], dstx_v, sem_b)
    pltpu.sync_copy(zacc_hbm, rows_a)
    pltpu.sync_copy(zdeg_hbm, deg_local)
    for j in range(ROWS_PER_TILE // CHUNK):
        pltpu.sync_copy(rows_a, acc_sh.at[pl.ds(row0 + j * CHUNK, CHUNK)])
    cp_f.wait()
    cp_d.wait()
    plsc.subcore_barrier()

    for k in range(nbuf):
        pltpu.async_copy(xw_hbm.at[fidx_v.at[k]], bufs[k], sems[k])

    @pl.loop(0, CHUNKS_PER_TILE, step=nbuf)
    def _quad(i):
        for k in range(nbuf):
            def _part(k=k):
                j = i + k
                buf, sem = bufs[k], sems[k]
                pltpu.make_async_copy(xw_hbm.at[fidx_v.at[j]], buf, sem).wait()
                pltpu.async_copy(buf, acc_sh.at[dstx_v.at[j]], sem_s, add=True)

                @pl.when(c == (k % 2))
                def _deg():
                    for l in range(CHUNK // 16):
                        idx16 = dstx_v[j, pl.ds(l * 16, 16)]
                        plsc.addupdate_scatter(deg_local, [idx16], ones16)

                pltpu.make_async_copy(buf, acc_sh.at[dstx_v.at[j]], sem_s).wait()

                @pl.when(j + nbuf < CHUNKS_PER_TILE)
                def _next():
                    pltpu.async_copy(xw_hbm.at[fidx_v.at[j + nbuf]], buf, sem)

            if k == 0:
                _part()
            else:
                pl.when(i + k < CHUNKS_PER_TILE)(_part)

    plsc.subcore_barrier()

    out_off = c * ACC_ROWS + row0
    for j in range(ROWS_PER_TILE // CHUNK):
        pltpu.sync_copy(acc_sh.at[pl.ds(row0 + j * CHUNK, CHUNK)], bufs[j % nbuf])
        pltpu.sync_copy(bufs[j % nbuf], pacc_hbm.at[pl.ds(out_off + j * CHUNK, CHUNK)])
    pltpu.sync_copy(deg_local, pdeg_hbm.at[c * NS + s])


def kernel(edge_index, edge_type, embed_table, bases1, w_comp1, bases2, w_comp2):
    f32 = jnp.float32
    i32 = jnp.int32
    src = edge_index[0].astype(i32)
    dst = edge_index[1].astype(i32)
    et = edge_type.astype(i32)
    pad = E_PAD - N_EDGES
    src_p = jnp.concatenate([src, jnp.zeros((pad,), i32)]).reshape(E_PAD // 128, 128)
    et_p = jnp.concatenate([et, jnp.zeros((pad,), i32)]).reshape(E_PAD // 128, 128)
    dst_p = jnp.concatenate([dst, jnp.full((pad,), N_NODES, i32)]).reshape(
        NS, CHUNKS_PER_TILE, CHUNK)

    flat2 = pl.pallas_call(
        _flat_body,
        grid=(),
        in_specs=[pl.BlockSpec((E_PAD // 128, 128), lambda: (0, 0)),
                  pl.BlockSpec((E_PAD // 128, 128), lambda: (0, 0))],
        out_specs=pl.BlockSpec((NC * E_PAD // 128, 128), lambda: (0, 0)),
        out_shape=jax.ShapeDtypeStruct((NC * E_PAD // 128, 128), i32),
    )(src_p, et_p).reshape(NC, NS, CHUNKS_PER_TILE, CHUNK)

    xw = pl.pallas_call(
        _xw_body,
        grid=(R,),
        in_specs=[pl.BlockSpec(memory_space=pltpu.SMEM),
                  pl.BlockSpec((4, H, H), lambda r: (0, 0, 0)),
                  pl.BlockSpec((N_NODES, H), lambda r: (0, 0))],
        out_specs=pl.BlockSpec((1, N_NODES, H), lambda r: (r, 0, 0)),
        out_shape=jax.ShapeDtypeStruct((R, N_NODES, H), f32),
    )(w_comp2, bases2, embed_table).reshape(NC * R * N_NODES, HC)

    zacc = jnp.zeros((CHUNK, HC), f32)
    zdeg = jnp.zeros((ACC_ROWS,), f32)

    mesh = plsc.VectorSubcoreMesh(core_axis_name="c", subcore_axis_name="s")
    pacc, pdeg = pl.kernel(
        _sc_body,
        out_type=(jax.ShapeDtypeStruct((NC * ACC_ROWS, HC), f32),
                  jax.ShapeDtypeStruct((NC * NS, ACC_ROWS), f32)),
        mesh=mesh,
        compiler_params=pltpu.CompilerParams(use_tc_tiling_on_sc=False,
                                             needs_layout_passes=False),
        scratch_types=[
            pltpu.VMEM((CHUNKS_PER_TILE, CHUNK), i32),
            pltpu.VMEM((CHUNKS_PER_TILE, CHUNK), i32),
            pltpu.VMEM((CHUNK, HC), f32),
            pltpu.VMEM((CHUNK, HC), f32),
            pltpu.VMEM((CHUNK, HC), f32),
            pltpu.VMEM((CHUNK, HC), f32),
            pltpu.VMEM((ACC_ROWS,), f32),
            pltpu.VMEM_SHARED((ACC_ROWS, HC), f32),
            pltpu.SemaphoreType.DMA,
            pltpu.SemaphoreType.DMA,
            pltpu.SemaphoreType.DMA,
            pltpu.SemaphoreType.DMA,
            pltpu.SemaphoreType.DMA,
        ],
    )(xw, flat2, dst_p, zacc, zdeg)

    pacc = pacc.reshape(NC, ACC_ROWS, HC)

    out = pl.pallas_call(
        _norm_body,
        grid=(),
        in_specs=[pl.BlockSpec((NC, ACC_ROWS, HC), lambda: (0, 0, 0)),
                  pl.BlockSpec((NC * NS, ACC_ROWS), lambda: (0, 0))],
        out_specs=pl.BlockSpec((N_NODES, H), lambda: (0, 0)),
        out_shape=jax.ShapeDtypeStruct((N_NODES, H), f32),
    )(pacc, pdeg)
    return out
